# baseline (speedup 1.0000x reference)
import functools

import jax
import jax.numpy as jnp
from jax import lax
from jax.experimental import pallas as pl
from jax.experimental.pallas import tpu as pltpu
from jax.experimental.pallas import tpu_sc as plsc

N = 10000
E = 160000
D = 256
ED = 16
B = 64

NW = 32
OWN = 320
NPAD = NW * OWN
G = 64
PADE = E + NW * G
TRASH = OWN
RB = 1000
RGRID = E // RB
CHUNK = 128
NCHUNKS = E // CHUNK
MAGIC = 6554


def _div320(d):
    return (d * MAGIC) >> 21


EB = 2000


def _elin_body(ea_ref, we1_ref, be1_ref, we2_ref, be2_ref, e1_ref, e2_ref):
    a = ea_ref[...]
    e1_ref[...] = jnp.dot(a, we1_ref[...],
                          preferred_element_type=jnp.float32) + be1_ref[...]
    e2_ref[...] = jnp.dot(a, we2_ref[...],
                          preferred_element_type=jnp.float32) + be2_ref[...]


def _elin2(edge_attr, We1, be1, We2, be2):
    return pl.pallas_call(
        _elin_body,
        grid=(E // EB,),
        in_specs=[
            pl.BlockSpec((EB, ED), lambda i: (i, 0)),
            pl.BlockSpec((ED, D), lambda i: (0, 0)),
            pl.BlockSpec((1, D), lambda i: (0, 0)),
            pl.BlockSpec((ED, D), lambda i: (0, 0)),
            pl.BlockSpec((1, D), lambda i: (0, 0)),
        ],
        out_specs=[
            pl.BlockSpec((EB, D), lambda i: (i, 0)),
            pl.BlockSpec((EB, D), lambda i: (i, 0)),
        ],
        out_shape=[
            jax.ShapeDtypeStruct((E, D), jnp.float32),
            jax.ShapeDtypeStruct((E, D), jnp.float32),
        ],
    )(edge_attr, We1, be1.reshape(1, D), We2, be2.reshape(1, D))


def _rank_body(dst_ref, intra_ref, base_ref, cnt_ref, run):
    i = pl.program_id(0)

    @pl.when(i == 0)
    def _():
        run[...] = jnp.zeros_like(run)

    dstv = dst_ref[0, 0, :]
    owner = _div320(dstv)
    seg = lax.broadcasted_iota(jnp.int32, (RB, NW), 1)
    m = jnp.where(owner[:, None] == seg, 1.0, 0.0).astype(jnp.float32)
    x = m
    sh = 1
    while sh < RB:
        x = x + jnp.concatenate(
            [jnp.zeros((sh, NW), jnp.float32), x[:-sh, :]], axis=0)
        sh *= 2
    excl = x - m
    intra = jnp.sum((run[...] + excl) * m, axis=1)
    intra_ref[0, 0, :] = intra.astype(jnp.int32)
    run[...] = run[...] + x[RB - 1:RB, :]

    cnts = run[...]
    padded = (jnp.floor((cnts + (G - 1)) / G) * G).astype(jnp.int32)
    tri = lax.broadcasted_iota(jnp.int32, (NW, NW), 0) < \
        lax.broadcasted_iota(jnp.int32, (NW, NW), 1)
    contrib = jnp.where(tri, jnp.broadcast_to(padded.reshape(NW, 1),
                                              (NW, NW)), 0)
    base = jnp.sum(contrib, axis=0)
    base_ref[...] = jnp.broadcast_to(base.reshape(NW, 1), (NW, 16))
    cnt_ref[...] = jnp.broadcast_to(cnts.astype(jnp.int32).reshape(NW, 1),
                                    (NW, 16))


def _rank(dst):
    dst3 = dst.reshape(RGRID, 1, RB)
    return pl.pallas_call(
        _rank_body,
        grid=(RGRID,),
        in_specs=[pl.BlockSpec((1, 1, RB), lambda i: (i, 0, 0))],
        out_specs=[
            pl.BlockSpec((1, 1, RB), lambda i: (i, 0, 0)),
            pl.BlockSpec((NW, 16), lambda i: (0, 0)),
            pl.BlockSpec((NW, 16), lambda i: (0, 0)),
        ],
        out_shape=[
            jax.ShapeDtypeStruct((RGRID, 1, RB), jnp.int32),
            jax.ShapeDtypeStruct((NW, 16), jnp.int32),
            jax.ShapeDtypeStruct((NW, 16), jnp.int32),
        ],
        scratch_shapes=[pltpu.VMEM((1, NW), jnp.float32)],
    )(dst3)


def _gloc_body(dst_ref, intra_ref, base_ref, g_ref, loc_ref):
    dstv = dst_ref[0, 0, :]
    owner = _div320(dstv)
    loc_ref[0, 0, :] = dstv - owner * OWN
    seg = lax.broadcasted_iota(jnp.int32, (RB, NW), 1)
    brow = base_ref[:, 0].reshape(1, NW)
    bv = jnp.sum(jnp.where(owner[:, None] == seg,
                           jnp.broadcast_to(brow, (RB, NW)), 0), axis=1)
    g_ref[0, 0, :] = bv + intra_ref[0, 0, :]


def _gloc(dst3, intra3, base2):
    return pl.pallas_call(
        _gloc_body,
        grid=(RGRID,),
        in_specs=[
            pl.BlockSpec((1, 1, RB), lambda i: (i, 0, 0)),
            pl.BlockSpec((1, 1, RB), lambda i: (i, 0, 0)),
            pl.BlockSpec((NW, 16), lambda i: (0, 0)),
        ],
        out_specs=[
            pl.BlockSpec((1, 1, RB), lambda i: (i, 0, 0)),
            pl.BlockSpec((1, 1, RB), lambda i: (i, 0, 0)),
        ],
        out_shape=[
            jax.ShapeDtypeStruct((RGRID, 1, RB), jnp.int32),
            jax.ShapeDtypeStruct((RGRID, 1, RB), jnp.int32),
        ],
    )(dst3, intra3, base2)


def _perm_body(src_hbm, g_hbm, loc_hbm, eid_hbm, sp_hbm,
               ep_hbm, lp_hbm, srcb, gbuf, lbuf, eidb):
    c = lax.axis_index("c")
    s = lax.axis_index("s")
    w = s * 2 + c

    my_lo = (w * NCHUNKS) >> 5
    my_hi = ((w + 1) * NCHUNKS) >> 5

    def chunk(ch, carry):
        ebase = ch * CHUNK
        pltpu.sync_copy(src_hbm.at[pl.ds(ebase, CHUNK)], srcb)
        pltpu.sync_copy(g_hbm.at[pl.ds(ebase, CHUNK)], gbuf)
        pltpu.sync_copy(loc_hbm.at[pl.ds(ebase, CHUNK)], lbuf)
        pltpu.sync_copy(eid_hbm.at[pl.ds(ebase, CHUNK)], eidb)
        pltpu.sync_copy(srcb, sp_hbm.at[gbuf])
        pltpu.sync_copy(eidb, ep_hbm.at[gbuf])
        pltpu.sync_copy(lbuf, lp_hbm.at[gbuf])
        return carry

    lax.fori_loop(my_lo, my_hi, chunk, 0)


_perm_call = pl.kernel(
    _perm_body,
    out_type=[
        jax.ShapeDtypeStruct((PADE,), jnp.int32),
        jax.ShapeDtypeStruct((PADE,), jnp.int32),
        jax.ShapeDtypeStruct((PADE,), jnp.int32),
    ],
    mesh=plsc.VectorSubcoreMesh(core_axis_name="c", subcore_axis_name="s"),
    scratch_types=[
        pltpu.VMEM((CHUNK,), jnp.int32),
        pltpu.VMEM((CHUNK,), jnp.int32),
        pltpu.VMEM((CHUNK,), jnp.int32),
        pltpu.VMEM((CHUNK,), jnp.int32),
    ],
)


def _msg_body(x_hbm, e_hbm, sp_hbm, ep_hbm, lp_hbm, base_hbm, cnt_hbm,
              out_hbm, sb, ebi, lb, basev, cntv, xbuf, ebuf, accum):
    c = lax.axis_index("c")
    s = lax.axis_index("s")
    w = s * 2 + c
    iota = lax.iota(jnp.int32, 16)
    zero16 = jnp.zeros((16,), jnp.float32)

    def zrow(r, cc):
        for k in range(16):
            accum[r, pl.ds(k * 16, 16)] = zero16
        return cc

    lax.fori_loop(0, OWN + 8, zrow, 0)

    pltpu.sync_copy(base_hbm.at[w], basev)
    pltpu.sync_copy(cnt_hbm.at[w], cntv)
    cnt = cntv[pl.ds(0, 16)][0]
    boff = pl.multiple_of(basev[pl.ds(0, 16)][0], G)
    nb = (cnt + (G - 1)) >> 6

    def batch(i, carry):
        off = pl.multiple_of(boff + i * G, G)
        pltpu.sync_copy(sp_hbm.at[pl.ds(off, G)], sb)
        pltpu.sync_copy(ep_hbm.at[pl.ds(off, G)], ebi)
        pltpu.sync_copy(lp_hbm.at[pl.ds(off, G)], lb)

        def sanitize(k, cc):
            valid = iota < (cnt - i * G - k * 16)
            sb[pl.ds(k * 16, 16)] = jnp.where(valid, sb[pl.ds(k * 16, 16)], 0)
            ebi[pl.ds(k * 16, 16)] = jnp.where(valid, ebi[pl.ds(k * 16, 16)],
                                               0)
            lb[pl.ds(k * 16, 16)] = jnp.where(valid, lb[pl.ds(k * 16, 16)],
                                              TRASH)
            return cc

        lax.fori_loop(0, G // 16, sanitize, 0)
        pltpu.sync_copy(x_hbm.at[sb], xbuf)
        pltpu.sync_copy(e_hbm.at[ebi], ebuf)

        def grp(k, cc):
            locv = lb[pl.ds(k * 16, 16)]
            for rr in range(16):
                l = locv[rr]
                r = k * 16 + rr
                for kk in range(16):
                    sl = pl.ds(kk * 16, 16)
                    m = jnp.maximum(xbuf[r, sl] + ebuf[r, sl], 0.0)
                    accum[l, sl] = accum[l, sl] + m
            return cc

        lax.fori_loop(0, G // 16, grp, 0)
        return carry

    lax.fori_loop(0, nb, batch, 0)
    pltpu.sync_copy(accum.at[pl.ds(0, OWN)], out_hbm.at[pl.ds(w * OWN, OWN)])


_msg_call = pl.kernel(
    _msg_body,
    out_type=jax.ShapeDtypeStruct((NPAD, D), jnp.float32),
    mesh=plsc.VectorSubcoreMesh(core_axis_name="c", subcore_axis_name="s"),
    scratch_types=[
        pltpu.VMEM((G,), jnp.int32),
        pltpu.VMEM((G,), jnp.int32),
        pltpu.VMEM((G,), jnp.int32),
        pltpu.VMEM((16,), jnp.int32),
        pltpu.VMEM((16,), jnp.int32),
        pltpu.VMEM((G, D), jnp.float32),
        pltpu.VMEM((G, D), jnp.float32),
        pltpu.VMEM((OWN + 8, D), jnp.float32),
    ],
)


NB = 400


def _node_body(relu, eps_ref, x_ref, agg_ref, w_ref, b_ref, h_ref):
    scale = 1.0 + eps_ref[0]
    pre = scale * x_ref[...] + agg_ref[...]
    h = jnp.dot(pre, w_ref[...],
                preferred_element_type=jnp.float32) + b_ref[...]
    if relu:
        h = jnp.maximum(h, 0.0)
    h_ref[...] = h


def _node_update(x, agg_pad, W, b, eps, relu):
    return pl.pallas_call(
        functools.partial(_node_body, relu),
        grid=(N // NB,),
        in_specs=[
            pl.BlockSpec(memory_space=pltpu.SMEM),
            pl.BlockSpec((NB, D), lambda i: (i, 0)),
            pl.BlockSpec((NB, D), lambda i: (i, 0)),
            pl.BlockSpec((D, D), lambda i: (0, 0)),
            pl.BlockSpec((1, D), lambda i: (0, 0)),
        ],
        out_specs=pl.BlockSpec((NB, D), lambda i: (i, 0)),
        out_shape=jax.ShapeDtypeStruct((N, D), jnp.float32),
    )(eps.reshape(1), x, agg_pad, W, b.reshape(1, D))


PB = 400


def _pool_body(h_ref, bi_ref, wh1_ref, bh1_ref, wh2_ref, bh2_ref, out_ref,
               sums, cnts):
    i = pl.program_id(0)

    @pl.when(i == 0)
    def _():
        sums[...] = jnp.zeros_like(sums)
        cnts[...] = jnp.zeros_like(cnts)

    bidx = bi_ref[0, 0, :]
    seg = lax.broadcasted_iota(jnp.int32, (B, PB), 0)
    maskf = jnp.where(bidx[None, :] == seg, 1.0, 0.0).astype(jnp.float32)
    sums[...] += jnp.dot(maskf, h_ref[...],
                         preferred_element_type=jnp.float32)
    cnts[...] += jnp.sum(maskf, axis=1, keepdims=True)

    @pl.when(i == pl.num_programs(0) - 1)
    def _():
        pooled = sums[...] / jnp.maximum(cnts[...], 1.0)
        z = jnp.maximum(
            jnp.dot(pooled, wh1_ref[...],
                    preferred_element_type=jnp.float32) + bh1_ref[...], 0.0)
        logits = jnp.dot(z, wh2_ref[...],
                         preferred_element_type=jnp.float32) + bh2_ref[...]
        out_ref[...] = jax.nn.sigmoid(logits)


def _pool_head(h, batch_idx, Wh1, bh1, Wh2, bh2):
    grid = N // PB
    bi = batch_idx.reshape(grid, 1, PB)
    return pl.pallas_call(
        _pool_body,
        grid=(grid,),
        in_specs=[
            pl.BlockSpec((PB, D), lambda i: (i, 0)),
            pl.BlockSpec((1, 1, PB), lambda i: (i, 0, 0)),
            pl.BlockSpec((D, 10), lambda i: (0, 0)),
            pl.BlockSpec((1, 10), lambda i: (0, 0)),
            pl.BlockSpec((10, 1), lambda i: (0, 0)),
            pl.BlockSpec((1, 1), lambda i: (0, 0)),
        ],
        out_specs=pl.BlockSpec((B, 1), lambda i: (0, 0)),
        out_shape=jax.ShapeDtypeStruct((B, 1), jnp.float32),
        scratch_shapes=[
            pltpu.VMEM((B, D), jnp.float32),
            pltpu.VMEM((B, 1), jnp.float32),
        ],
    )(h, bi, Wh1, bh1.reshape(1, 10), Wh2, bh2.reshape(1, 1))


def kernel(x, edge_index, edge_attr, batch_idx, W1, b1, We1, be1, eps1,
           W2, b2, We2, be2, eps2, Wh1, bh1, Wh2, bh2):
    src = edge_index[0]
    dst = edge_index[1]
    e1, e2 = _elin2(edge_attr, We1, be1, We2, be2)
    dst3 = dst.reshape(RGRID, 1, RB)
    intra3, base2, cnt2 = _rank(dst)
    g3, loc3 = _gloc(dst3, intra3, base2)
    sp, ep, lp = _perm_call(src, g3.reshape(E), loc3.reshape(E),
                            jnp.arange(E, dtype=jnp.int32))
    agg1 = _msg_call(x, e1, sp, ep, lp, base2, cnt2)
    h1 = _node_update(x, agg1, W1, b1, eps1, relu=True)
    agg2 = _msg_call(h1, e2, sp, ep, lp, base2, cnt2)
    h2 = _node_update(h1, agg2, W2, b2, eps2, relu=False)
    return _pool_head(h2, batch_idx, Wh1, bh1, Wh2, bh2)


# speedup vs baseline: 1.0513x; 1.0513x over previous
import functools

import jax
import jax.numpy as jnp
from jax import lax
from jax.experimental import pallas as pl
from jax.experimental.pallas import tpu as pltpu
from jax.experimental.pallas import tpu_sc as plsc

N = 10000
E = 160000
D = 256
ED = 16
B = 64

NW = 32
OWN = 320
NPAD = NW * OWN
G = 64
PADE = E + NW * G
TRASH = OWN
RB = 1000
RGRID = E // RB
CHUNK = 128
NCHUNKS = E // CHUNK
MAGIC = 6554


def _div320(d):
    return (d * MAGIC) >> 21


EB = 2000


def _elin_body(ea_ref, we1_ref, be1_ref, we2_ref, be2_ref, e1_ref, e2_ref):
    a = ea_ref[...]
    e1_ref[...] = jnp.dot(a, we1_ref[...],
                          preferred_element_type=jnp.float32) + be1_ref[...]
    e2_ref[...] = jnp.dot(a, we2_ref[...],
                          preferred_element_type=jnp.float32) + be2_ref[...]


def _elin2(edge_attr, We1, be1, We2, be2):
    return pl.pallas_call(
        _elin_body,
        grid=(E // EB,),
        in_specs=[
            pl.BlockSpec((EB, ED), lambda i: (i, 0)),
            pl.BlockSpec((ED, D), lambda i: (0, 0)),
            pl.BlockSpec((1, D), lambda i: (0, 0)),
            pl.BlockSpec((ED, D), lambda i: (0, 0)),
            pl.BlockSpec((1, D), lambda i: (0, 0)),
        ],
        out_specs=[
            pl.BlockSpec((EB, D), lambda i: (i, 0)),
            pl.BlockSpec((EB, D), lambda i: (i, 0)),
        ],
        out_shape=[
            jax.ShapeDtypeStruct((E, D), jnp.float32),
            jax.ShapeDtypeStruct((E, D), jnp.float32),
        ],
    )(edge_attr, We1, be1.reshape(1, D), We2, be2.reshape(1, D))


def _rank_body(dst_ref, intra_ref, base_ref, cnt_ref, run):
    i = pl.program_id(0)

    @pl.when(i == 0)
    def _():
        run[...] = jnp.zeros_like(run)

    dstv = dst_ref[0, 0, :]
    owner = _div320(dstv)
    seg = lax.broadcasted_iota(jnp.int32, (RB, NW), 1)
    m = jnp.where(owner[:, None] == seg, 1.0, 0.0).astype(jnp.float32)
    x = m
    sh = 1
    while sh < RB:
        x = x + jnp.concatenate(
            [jnp.zeros((sh, NW), jnp.float32), x[:-sh, :]], axis=0)
        sh *= 2
    excl = x - m
    intra = jnp.sum((run[...] + excl) * m, axis=1)
    intra_ref[0, 0, :] = intra.astype(jnp.int32)
    run[...] = run[...] + x[RB - 1:RB, :]

    cnts = run[...]
    padded = (jnp.floor((cnts + (G - 1)) / G) * G).astype(jnp.int32)
    tri = lax.broadcasted_iota(jnp.int32, (NW, NW), 0) < \
        lax.broadcasted_iota(jnp.int32, (NW, NW), 1)
    contrib = jnp.where(tri, jnp.broadcast_to(padded.reshape(NW, 1),
                                              (NW, NW)), 0)
    base = jnp.sum(contrib, axis=0)
    base_ref[...] = jnp.broadcast_to(base.reshape(NW, 1), (NW, 16))
    cnt_ref[...] = jnp.broadcast_to(cnts.astype(jnp.int32).reshape(NW, 1),
                                    (NW, 16))


def _rank(dst):
    dst3 = dst.reshape(RGRID, 1, RB)
    return pl.pallas_call(
        _rank_body,
        grid=(RGRID,),
        in_specs=[pl.BlockSpec((1, 1, RB), lambda i: (i, 0, 0))],
        out_specs=[
            pl.BlockSpec((1, 1, RB), lambda i: (i, 0, 0)),
            pl.BlockSpec((NW, 16), lambda i: (0, 0)),
            pl.BlockSpec((NW, 16), lambda i: (0, 0)),
        ],
        out_shape=[
            jax.ShapeDtypeStruct((RGRID, 1, RB), jnp.int32),
            jax.ShapeDtypeStruct((NW, 16), jnp.int32),
            jax.ShapeDtypeStruct((NW, 16), jnp.int32),
        ],
        scratch_shapes=[pltpu.VMEM((1, NW), jnp.float32)],
    )(dst3)


def _gloc_body(dst_ref, intra_ref, base_ref, g_ref, loc_ref):
    dstv = dst_ref[0, 0, :]
    owner = _div320(dstv)
    loc_ref[0, 0, :] = dstv - owner * OWN
    seg = lax.broadcasted_iota(jnp.int32, (RB, NW), 1)
    brow = base_ref[:, 0].reshape(1, NW)
    bv = jnp.sum(jnp.where(owner[:, None] == seg,
                           jnp.broadcast_to(brow, (RB, NW)), 0), axis=1)
    g_ref[0, 0, :] = bv + intra_ref[0, 0, :]


def _gloc(dst3, intra3, base2):
    return pl.pallas_call(
        _gloc_body,
        grid=(RGRID,),
        in_specs=[
            pl.BlockSpec((1, 1, RB), lambda i: (i, 0, 0)),
            pl.BlockSpec((1, 1, RB), lambda i: (i, 0, 0)),
            pl.BlockSpec((NW, 16), lambda i: (0, 0)),
        ],
        out_specs=[
            pl.BlockSpec((1, 1, RB), lambda i: (i, 0, 0)),
            pl.BlockSpec((1, 1, RB), lambda i: (i, 0, 0)),
        ],
        out_shape=[
            jax.ShapeDtypeStruct((RGRID, 1, RB), jnp.int32),
            jax.ShapeDtypeStruct((RGRID, 1, RB), jnp.int32),
        ],
    )(dst3, intra3, base2)


def _perm_body(src_hbm, g_hbm, loc_hbm, eid_hbm, sp_hbm,
               ep_hbm, lp_hbm, srcb, gbuf, lbuf, eidb):
    c = lax.axis_index("c")
    s = lax.axis_index("s")
    w = s * 2 + c

    my_lo = (w * NCHUNKS) >> 5
    my_hi = ((w + 1) * NCHUNKS) >> 5

    def chunk(ch, carry):
        ebase = ch * CHUNK
        pltpu.sync_copy(src_hbm.at[pl.ds(ebase, CHUNK)], srcb)
        pltpu.sync_copy(g_hbm.at[pl.ds(ebase, CHUNK)], gbuf)
        pltpu.sync_copy(loc_hbm.at[pl.ds(ebase, CHUNK)], lbuf)
        pltpu.sync_copy(eid_hbm.at[pl.ds(ebase, CHUNK)], eidb)
        pltpu.sync_copy(srcb, sp_hbm.at[gbuf])
        pltpu.sync_copy(eidb, ep_hbm.at[gbuf])
        pltpu.sync_copy(lbuf, lp_hbm.at[gbuf])
        return carry

    lax.fori_loop(my_lo, my_hi, chunk, 0)


_perm_call = pl.kernel(
    _perm_body,
    out_type=[
        jax.ShapeDtypeStruct((PADE + 1024,), jnp.int32),
        jax.ShapeDtypeStruct((PADE + 1024,), jnp.int32),
        jax.ShapeDtypeStruct((PADE + 1024,), jnp.int32),
    ],
    mesh=plsc.VectorSubcoreMesh(core_axis_name="c", subcore_axis_name="s"),
    scratch_types=[
        pltpu.VMEM((CHUNK,), jnp.int32),
        pltpu.VMEM((CHUNK,), jnp.int32),
        pltpu.VMEM((CHUNK,), jnp.int32),
        pltpu.VMEM((CHUNK,), jnp.int32),
    ],
)


GB = 64
SB = 1024


def _msg_body(x_hbm, e_hbm, sp_hbm, ep_hbm, lp_hbm, base_hbm, cnt_hbm,
              out_hbm, spb, epb, lpb, basev, cntv, xbuf, ebuf, accum):
    c = lax.axis_index("c")
    s = lax.axis_index("s")
    w = s * 2 + c
    iota = lax.iota(jnp.int32, 16)
    zero16 = jnp.zeros((16,), jnp.float32)

    def zrow(r, cc):
        for k in range(16):
            accum[r, pl.ds(k * 16, 16)] = zero16
        return cc

    lax.fori_loop(0, OWN + 8, zrow, 0)

    pltpu.sync_copy(base_hbm.at[w], basev)
    pltpu.sync_copy(cnt_hbm.at[w], cntv)
    cnt = cntv[pl.ds(0, 16)][0]
    boff = pl.multiple_of(basev[pl.ds(0, 16)][0], G)
    nb = (cnt + (GB - 1)) >> 6
    nsb = (cnt + (SB - 1)) >> 10

    def super_batch(i, carry):
        off = pl.multiple_of(boff + i * SB, G)
        pltpu.sync_copy(sp_hbm.at[pl.ds(off, SB)], spb)
        pltpu.sync_copy(ep_hbm.at[pl.ds(off, SB)], epb)
        pltpu.sync_copy(lp_hbm.at[pl.ds(off, SB)], lpb)
        jn = jnp.minimum(SB // GB, nb - i * (SB // GB))

        def batch(j, cc):
            def sanitize(k, c2):
                valid = iota < (cnt - i * SB - j * GB - k * 16)
                sl = pl.ds(j * GB + k * 16, 16)
                spb[sl] = jnp.where(valid, spb[sl], 0)
                epb[sl] = jnp.where(valid, epb[sl], 0)
                lpb[sl] = jnp.where(valid, lpb[sl], TRASH)
                return c2

            lax.fori_loop(0, GB // 16, sanitize, 0)
            pltpu.sync_copy(x_hbm.at[spb.at[pl.ds(j * GB, GB)]], xbuf)
            pltpu.sync_copy(e_hbm.at[epb.at[pl.ds(j * GB, GB)]], ebuf)

            def grp(k, c2):
                locv = lpb[pl.ds(j * GB + k * 16, 16)]
                for rr in range(16):
                    l = locv[rr]
                    r = k * 16 + rr
                    for kk in range(16):
                        sl = pl.ds(kk * 16, 16)
                        m = jnp.maximum(xbuf[r, sl] + ebuf[r, sl], 0.0)
                        accum[l, sl] = accum[l, sl] + m
                return c2

            lax.fori_loop(0, GB // 16, grp, 0)
            return cc

        lax.fori_loop(0, jn, batch, 0)
        return carry

    lax.fori_loop(0, nsb, super_batch, 0)
    pltpu.sync_copy(accum.at[pl.ds(0, OWN)], out_hbm.at[pl.ds(w * OWN, OWN)])


_msg_call = pl.kernel(
    _msg_body,
    out_type=jax.ShapeDtypeStruct((NPAD, D), jnp.float32),
    mesh=plsc.VectorSubcoreMesh(core_axis_name="c", subcore_axis_name="s"),
    scratch_types=[
        pltpu.VMEM((SB,), jnp.int32),
        pltpu.VMEM((SB,), jnp.int32),
        pltpu.VMEM((SB,), jnp.int32),
        pltpu.VMEM((16,), jnp.int32),
        pltpu.VMEM((16,), jnp.int32),
        pltpu.VMEM((GB, D), jnp.float32),
        pltpu.VMEM((GB, D), jnp.float32),
        pltpu.VMEM((OWN + 8, D), jnp.float32),
    ],
)


NB = 400


def _node_body(relu, eps_ref, x_ref, agg_ref, w_ref, b_ref, h_ref):
    scale = 1.0 + eps_ref[0]
    pre = scale * x_ref[...] + agg_ref[...]
    h = jnp.dot(pre, w_ref[...],
                preferred_element_type=jnp.float32) + b_ref[...]
    if relu:
        h = jnp.maximum(h, 0.0)
    h_ref[...] = h


def _node_update(x, agg_pad, W, b, eps, relu):
    return pl.pallas_call(
        functools.partial(_node_body, relu),
        grid=(N // NB,),
        in_specs=[
            pl.BlockSpec(memory_space=pltpu.SMEM),
            pl.BlockSpec((NB, D), lambda i: (i, 0)),
            pl.BlockSpec((NB, D), lambda i: (i, 0)),
            pl.BlockSpec((D, D), lambda i: (0, 0)),
            pl.BlockSpec((1, D), lambda i: (0, 0)),
        ],
        out_specs=pl.BlockSpec((NB, D), lambda i: (i, 0)),
        out_shape=jax.ShapeDtypeStruct((N, D), jnp.float32),
    )(eps.reshape(1), x, agg_pad, W, b.reshape(1, D))


PB = 400


def _pool_body(h_ref, bi_ref, wh1_ref, bh1_ref, wh2_ref, bh2_ref, out_ref,
               sums, cnts):
    i = pl.program_id(0)

    @pl.when(i == 0)
    def _():
        sums[...] = jnp.zeros_like(sums)
        cnts[...] = jnp.zeros_like(cnts)

    bidx = bi_ref[0, 0, :]
    seg = lax.broadcasted_iota(jnp.int32, (B, PB), 0)
    maskf = jnp.where(bidx[None, :] == seg, 1.0, 0.0).astype(jnp.float32)
    sums[...] += jnp.dot(maskf, h_ref[...],
                         preferred_element_type=jnp.float32)
    cnts[...] += jnp.sum(maskf, axis=1, keepdims=True)

    @pl.when(i == pl.num_programs(0) - 1)
    def _():
        pooled = sums[...] / jnp.maximum(cnts[...], 1.0)
        z = jnp.maximum(
            jnp.dot(pooled, wh1_ref[...],
                    preferred_element_type=jnp.float32) + bh1_ref[...], 0.0)
        logits = jnp.dot(z, wh2_ref[...],
                         preferred_element_type=jnp.float32) + bh2_ref[...]
        out_ref[...] = jax.nn.sigmoid(logits)


def _pool_head(h, batch_idx, Wh1, bh1, Wh2, bh2):
    grid = N // PB
    bi = batch_idx.reshape(grid, 1, PB)
    return pl.pallas_call(
        _pool_body,
        grid=(grid,),
        in_specs=[
            pl.BlockSpec((PB, D), lambda i: (i, 0)),
            pl.BlockSpec((1, 1, PB), lambda i: (i, 0, 0)),
            pl.BlockSpec((D, 10), lambda i: (0, 0)),
            pl.BlockSpec((1, 10), lambda i: (0, 0)),
            pl.BlockSpec((10, 1), lambda i: (0, 0)),
            pl.BlockSpec((1, 1), lambda i: (0, 0)),
        ],
        out_specs=pl.BlockSpec((B, 1), lambda i: (0, 0)),
        out_shape=jax.ShapeDtypeStruct((B, 1), jnp.float32),
        scratch_shapes=[
            pltpu.VMEM((B, D), jnp.float32),
            pltpu.VMEM((B, 1), jnp.float32),
        ],
    )(h, bi, Wh1, bh1.reshape(1, 10), Wh2, bh2.reshape(1, 1))


def kernel(x, edge_index, edge_attr, batch_idx, W1, b1, We1, be1, eps1,
           W2, b2, We2, be2, eps2, Wh1, bh1, Wh2, bh2):
    src = edge_index[0]
    dst = edge_index[1]
    e1, e2 = _elin2(edge_attr, We1, be1, We2, be2)
    dst3 = dst.reshape(RGRID, 1, RB)
    intra3, base2, cnt2 = _rank(dst)
    g3, loc3 = _gloc(dst3, intra3, base2)
    sp, ep, lp = _perm_call(src, g3.reshape(E), loc3.reshape(E),
                            jnp.arange(E, dtype=jnp.int32))
    agg1 = _msg_call(x, e1, sp, ep, lp, base2, cnt2)
    h1 = _node_update(x, agg1, W1, b1, eps1, relu=True)
    agg2 = _msg_call(h1, e2, sp, ep, lp, base2, cnt2)
    h2 = _node_update(h1, agg2, W2, b2, eps2, relu=False)
    return _pool_head(h2, batch_idx, Wh1, bh1, Wh2, bh2)


# speedup vs baseline: 1.0749x; 1.0224x over previous
import functools

import jax
import jax.numpy as jnp
from jax import lax
from jax.experimental import pallas as pl
from jax.experimental.pallas import tpu as pltpu
from jax.experimental.pallas import tpu_sc as plsc

N = 10000
E = 160000
D = 256
ED = 16
B = 64

NW = 32
OWN = 320
NPAD = NW * OWN
G = 64
PADE = E + NW * G
TRASH = OWN
RB = 1000
RGRID = E // RB
CHUNK = 128
NCHUNKS = E // CHUNK
MAGIC = 6554


def _div320(d):
    return (d * MAGIC) >> 21


EB = 2000


def _elin_body(ea_ref, we1_ref, be1_ref, we2_ref, be2_ref, e1_ref, e2_ref):
    a = ea_ref[...]
    e1_ref[...] = jnp.dot(a, we1_ref[...],
                          preferred_element_type=jnp.float32) + be1_ref[...]
    e2_ref[...] = jnp.dot(a, we2_ref[...],
                          preferred_element_type=jnp.float32) + be2_ref[...]


def _elin2(edge_attr, We1, be1, We2, be2):
    return pl.pallas_call(
        _elin_body,
        grid=(E // EB,),
        in_specs=[
            pl.BlockSpec((EB, ED), lambda i: (i, 0)),
            pl.BlockSpec((ED, D), lambda i: (0, 0)),
            pl.BlockSpec((1, D), lambda i: (0, 0)),
            pl.BlockSpec((ED, D), lambda i: (0, 0)),
            pl.BlockSpec((1, D), lambda i: (0, 0)),
        ],
        out_specs=[
            pl.BlockSpec((EB, D), lambda i: (i, 0)),
            pl.BlockSpec((EB, D), lambda i: (i, 0)),
        ],
        out_shape=[
            jax.ShapeDtypeStruct((E, D), jnp.float32),
            jax.ShapeDtypeStruct((E, D), jnp.float32),
        ],
    )(edge_attr, We1, be1.reshape(1, D), We2, be2.reshape(1, D))


def _rank_body(dst_ref, intra_ref, base_ref, cnt_ref, run):
    i = pl.program_id(0)

    @pl.when(i == 0)
    def _():
        run[...] = jnp.zeros_like(run)

    dstv = dst_ref[0, 0, :]
    owner = _div320(dstv)
    seg = lax.broadcasted_iota(jnp.int32, (RB, NW), 1)
    m = jnp.where(owner[:, None] == seg, 1.0, 0.0).astype(jnp.float32)
    x = m
    sh = 1
    while sh < RB:
        x = x + jnp.concatenate(
            [jnp.zeros((sh, NW), jnp.float32), x[:-sh, :]], axis=0)
        sh *= 2
    excl = x - m
    intra = jnp.sum((run[...] + excl) * m, axis=1)
    intra_ref[0, 0, :] = intra.astype(jnp.int32)
    run[...] = run[...] + x[RB - 1:RB, :]

    cnts = run[...]
    padded = (jnp.floor((cnts + (G - 1)) / G) * G).astype(jnp.int32)
    tri = lax.broadcasted_iota(jnp.int32, (NW, NW), 0) < \
        lax.broadcasted_iota(jnp.int32, (NW, NW), 1)
    contrib = jnp.where(tri, jnp.broadcast_to(padded.reshape(NW, 1),
                                              (NW, NW)), 0)
    base = jnp.sum(contrib, axis=0)
    base_ref[...] = jnp.broadcast_to(base.reshape(NW, 1), (NW, 16))
    cnt_ref[...] = jnp.broadcast_to(cnts.astype(jnp.int32).reshape(NW, 1),
                                    (NW, 16))


def _rank(dst):
    dst3 = dst.reshape(RGRID, 1, RB)
    return pl.pallas_call(
        _rank_body,
        grid=(RGRID,),
        in_specs=[pl.BlockSpec((1, 1, RB), lambda i: (i, 0, 0))],
        out_specs=[
            pl.BlockSpec((1, 1, RB), lambda i: (i, 0, 0)),
            pl.BlockSpec((NW, 16), lambda i: (0, 0)),
            pl.BlockSpec((NW, 16), lambda i: (0, 0)),
        ],
        out_shape=[
            jax.ShapeDtypeStruct((RGRID, 1, RB), jnp.int32),
            jax.ShapeDtypeStruct((NW, 16), jnp.int32),
            jax.ShapeDtypeStruct((NW, 16), jnp.int32),
        ],
        scratch_shapes=[pltpu.VMEM((1, NW), jnp.float32)],
    )(dst3)


def _gloc_body(dst_ref, intra_ref, base_ref, g_ref, loc_ref):
    dstv = dst_ref[0, 0, :]
    owner = _div320(dstv)
    loc_ref[0, 0, :] = dstv - owner * OWN
    seg = lax.broadcasted_iota(jnp.int32, (RB, NW), 1)
    brow = base_ref[:, 0].reshape(1, NW)
    bv = jnp.sum(jnp.where(owner[:, None] == seg,
                           jnp.broadcast_to(brow, (RB, NW)), 0), axis=1)
    g_ref[0, 0, :] = bv + intra_ref[0, 0, :]


def _gloc(dst3, intra3, base2):
    return pl.pallas_call(
        _gloc_body,
        grid=(RGRID,),
        in_specs=[
            pl.BlockSpec((1, 1, RB), lambda i: (i, 0, 0)),
            pl.BlockSpec((1, 1, RB), lambda i: (i, 0, 0)),
            pl.BlockSpec((NW, 16), lambda i: (0, 0)),
        ],
        out_specs=[
            pl.BlockSpec((1, 1, RB), lambda i: (i, 0, 0)),
            pl.BlockSpec((1, 1, RB), lambda i: (i, 0, 0)),
        ],
        out_shape=[
            jax.ShapeDtypeStruct((RGRID, 1, RB), jnp.int32),
            jax.ShapeDtypeStruct((RGRID, 1, RB), jnp.int32),
        ],
    )(dst3, intra3, base2)


def _perm_body(src_hbm, g_hbm, loc_hbm, eid_hbm, sp_hbm,
               ep_hbm, lp_hbm, srcb, gbuf, lbuf, eidb, sem1, sem2):
    c = lax.axis_index("c")
    s = lax.axis_index("s")
    w = s * 2 + c

    my_lo = (w * NCHUNKS) >> 5
    my_hi = ((w + 1) * NCHUNKS) >> 5

    def chunk(ch, carry):
        ebase = ch * CHUNK
        c1 = pltpu.async_copy(src_hbm.at[pl.ds(ebase, CHUNK)], srcb, sem1)
        c2 = pltpu.async_copy(g_hbm.at[pl.ds(ebase, CHUNK)], gbuf, sem1)
        c3 = pltpu.async_copy(loc_hbm.at[pl.ds(ebase, CHUNK)], lbuf, sem1)
        c4 = pltpu.async_copy(eid_hbm.at[pl.ds(ebase, CHUNK)], eidb, sem1)
        c1.wait()
        c2.wait()
        c3.wait()
        c4.wait()
        s1 = pltpu.async_copy(srcb, sp_hbm.at[gbuf], sem2)
        s2 = pltpu.async_copy(eidb, ep_hbm.at[gbuf], sem2)
        s3 = pltpu.async_copy(lbuf, lp_hbm.at[gbuf], sem2)
        s1.wait()
        s2.wait()
        s3.wait()
        return carry

    lax.fori_loop(my_lo, my_hi, chunk, 0)


_perm_call = pl.kernel(
    _perm_body,
    out_type=[
        jax.ShapeDtypeStruct((PADE + 1024,), jnp.int32),
        jax.ShapeDtypeStruct((PADE + 1024,), jnp.int32),
        jax.ShapeDtypeStruct((PADE + 1024,), jnp.int32),
    ],
    mesh=plsc.VectorSubcoreMesh(core_axis_name="c", subcore_axis_name="s"),
    scratch_types=[
        pltpu.VMEM((CHUNK,), jnp.int32),
        pltpu.VMEM((CHUNK,), jnp.int32),
        pltpu.VMEM((CHUNK,), jnp.int32),
        pltpu.VMEM((CHUNK,), jnp.int32),
        pltpu.SemaphoreType.DMA,
        pltpu.SemaphoreType.DMA,
    ],
)


GB = 64
SB = 1024


def _msg_body(x_hbm, e_hbm, sp_hbm, ep_hbm, lp_hbm, base_hbm, cnt_hbm,
              out_hbm, spb, epb, lpb, basev, cntv, xbuf, ebuf, accum,
              semx, seme):
    c = lax.axis_index("c")
    s = lax.axis_index("s")
    w = s * 2 + c
    iota = lax.iota(jnp.int32, 16)
    zero16 = jnp.zeros((16,), jnp.float32)

    def zrow(r, cc):
        for k in range(16):
            accum[r, pl.ds(k * 16, 16)] = zero16
        return cc

    lax.fori_loop(0, OWN + 8, zrow, 0)

    pltpu.sync_copy(base_hbm.at[w], basev)
    pltpu.sync_copy(cnt_hbm.at[w], cntv)
    cnt = cntv[pl.ds(0, 16)][0]
    boff = pl.multiple_of(basev[pl.ds(0, 16)][0], G)
    nb = (cnt + (GB - 1)) >> 6
    nsb = (cnt + (SB - 1)) >> 10

    def super_batch(i, carry):
        off = pl.multiple_of(boff + i * SB, G)
        pltpu.sync_copy(sp_hbm.at[pl.ds(off, SB)], spb)
        pltpu.sync_copy(ep_hbm.at[pl.ds(off, SB)], epb)
        pltpu.sync_copy(lp_hbm.at[pl.ds(off, SB)], lpb)
        jn = jnp.minimum(SB // GB, nb - i * (SB // GB))

        def batch(j, cc):
            def sanitize(k, c2):
                valid = iota < (cnt - i * SB - j * GB - k * 16)
                sl = pl.ds(j * GB + k * 16, 16)
                spb[sl] = jnp.where(valid, spb[sl], 0)
                epb[sl] = jnp.where(valid, epb[sl], 0)
                lpb[sl] = jnp.where(valid, lpb[sl], TRASH)
                return c2

            lax.fori_loop(0, GB // 16, sanitize, 0)
            cpx = pltpu.async_copy(x_hbm.at[spb.at[pl.ds(j * GB, GB)]],
                                   xbuf, semx)
            cpe = pltpu.async_copy(e_hbm.at[epb.at[pl.ds(j * GB, GB)]],
                                   ebuf, seme)
            cpx.wait()
            cpe.wait()

            def grp(k, c2):
                locv = lpb[pl.ds(j * GB + k * 16, 16)]
                for rr in range(16):
                    l = locv[rr]
                    r = k * 16 + rr
                    for kk in range(16):
                        sl = pl.ds(kk * 16, 16)
                        m = jnp.maximum(xbuf[r, sl] + ebuf[r, sl], 0.0)
                        accum[l, sl] = accum[l, sl] + m
                return c2

            lax.fori_loop(0, GB // 16, grp, 0)
            return cc

        lax.fori_loop(0, jn, batch, 0)
        return carry

    lax.fori_loop(0, nsb, super_batch, 0)
    pltpu.sync_copy(accum.at[pl.ds(0, OWN)], out_hbm.at[pl.ds(w * OWN, OWN)])


_msg_call = pl.kernel(
    _msg_body,
    out_type=jax.ShapeDtypeStruct((NPAD, D), jnp.float32),
    mesh=plsc.VectorSubcoreMesh(core_axis_name="c", subcore_axis_name="s"),
    scratch_types=[
        pltpu.VMEM((SB,), jnp.int32),
        pltpu.VMEM((SB,), jnp.int32),
        pltpu.VMEM((SB,), jnp.int32),
        pltpu.VMEM((16,), jnp.int32),
        pltpu.VMEM((16,), jnp.int32),
        pltpu.VMEM((GB, D), jnp.float32),
        pltpu.VMEM((GB, D), jnp.float32),
        pltpu.VMEM((OWN + 8, D), jnp.float32),
        pltpu.SemaphoreType.DMA,
        pltpu.SemaphoreType.DMA,
    ],
)


NB = 400


def _node_body(relu, eps_ref, x_ref, agg_ref, w_ref, b_ref, h_ref):
    scale = 1.0 + eps_ref[0]
    pre = scale * x_ref[...] + agg_ref[...]
    h = jnp.dot(pre, w_ref[...],
                preferred_element_type=jnp.float32) + b_ref[...]
    if relu:
        h = jnp.maximum(h, 0.0)
    h_ref[...] = h


def _node_update(x, agg_pad, W, b, eps, relu):
    return pl.pallas_call(
        functools.partial(_node_body, relu),
        grid=(N // NB,),
        in_specs=[
            pl.BlockSpec(memory_space=pltpu.SMEM),
            pl.BlockSpec((NB, D), lambda i: (i, 0)),
            pl.BlockSpec((NB, D), lambda i: (i, 0)),
            pl.BlockSpec((D, D), lambda i: (0, 0)),
            pl.BlockSpec((1, D), lambda i: (0, 0)),
        ],
        out_specs=pl.BlockSpec((NB, D), lambda i: (i, 0)),
        out_shape=jax.ShapeDtypeStruct((N, D), jnp.float32),
    )(eps.reshape(1), x, agg_pad, W, b.reshape(1, D))


PB = 400


def _pool_body(h_ref, bi_ref, wh1_ref, bh1_ref, wh2_ref, bh2_ref, out_ref,
               sums, cnts):
    i = pl.program_id(0)

    @pl.when(i == 0)
    def _():
        sums[...] = jnp.zeros_like(sums)
        cnts[...] = jnp.zeros_like(cnts)

    bidx = bi_ref[0, 0, :]
    seg = lax.broadcasted_iota(jnp.int32, (B, PB), 0)
    maskf = jnp.where(bidx[None, :] == seg, 1.0, 0.0).astype(jnp.float32)
    sums[...] += jnp.dot(maskf, h_ref[...],
                         preferred_element_type=jnp.float32)
    cnts[...] += jnp.sum(maskf, axis=1, keepdims=True)

    @pl.when(i == pl.num_programs(0) - 1)
    def _():
        pooled = sums[...] / jnp.maximum(cnts[...], 1.0)
        z = jnp.maximum(
            jnp.dot(pooled, wh1_ref[...],
                    preferred_element_type=jnp.float32) + bh1_ref[...], 0.0)
        logits = jnp.dot(z, wh2_ref[...],
                         preferred_element_type=jnp.float32) + bh2_ref[...]
        out_ref[...] = jax.nn.sigmoid(logits)


def _pool_head(h, batch_idx, Wh1, bh1, Wh2, bh2):
    grid = N // PB
    bi = batch_idx.reshape(grid, 1, PB)
    return pl.pallas_call(
        _pool_body,
        grid=(grid,),
        in_specs=[
            pl.BlockSpec((PB, D), lambda i: (i, 0)),
            pl.BlockSpec((1, 1, PB), lambda i: (i, 0, 0)),
            pl.BlockSpec((D, 10), lambda i: (0, 0)),
            pl.BlockSpec((1, 10), lambda i: (0, 0)),
            pl.BlockSpec((10, 1), lambda i: (0, 0)),
            pl.BlockSpec((1, 1), lambda i: (0, 0)),
        ],
        out_specs=pl.BlockSpec((B, 1), lambda i: (0, 0)),
        out_shape=jax.ShapeDtypeStruct((B, 1), jnp.float32),
        scratch_shapes=[
            pltpu.VMEM((B, D), jnp.float32),
            pltpu.VMEM((B, 1), jnp.float32),
        ],
    )(h, bi, Wh1, bh1.reshape(1, 10), Wh2, bh2.reshape(1, 1))


def kernel(x, edge_index, edge_attr, batch_idx, W1, b1, We1, be1, eps1,
           W2, b2, We2, be2, eps2, Wh1, bh1, Wh2, bh2):
    src = edge_index[0]
    dst = edge_index[1]
    e1, e2 = _elin2(edge_attr, We1, be1, We2, be2)
    dst3 = dst.reshape(RGRID, 1, RB)
    intra3, base2, cnt2 = _rank(dst)
    g3, loc3 = _gloc(dst3, intra3, base2)
    sp, ep, lp = _perm_call(src, g3.reshape(E), loc3.reshape(E),
                            jnp.arange(E, dtype=jnp.int32))
    agg1 = _msg_call(x, e1, sp, ep, lp, base2, cnt2)
    h1 = _node_update(x, agg1, W1, b1, eps1, relu=True)
    agg2 = _msg_call(h1, e2, sp, ep, lp, base2, cnt2)
    h2 = _node_update(h1, agg2, W2, b2, eps2, relu=False)
    return _pool_head(h2, batch_idx, Wh1, bh1, Wh2, bh2)
